# baseline (speedup 1.0000x reference)
import functools

import jax
import jax.numpy as jnp
from jax import lax
from jax.experimental import pallas as pl
from jax.experimental.pallas import tpu as pltpu
from jax.experimental.pallas import tpu_sc as plsc

_NC = 2
_NS = 16
_NW = _NC * _NS

_P = 50
_TI = 128
_NTT = 128
_D = 64
_CT = _D // 8
_TPW = _NTT // _NW
_UNITS = _TPW * _P


def _make_gather():
    mesh = plsc.VectorSubcoreMesh(core_axis_name="c", subcore_axis_name="s")

    @functools.partial(
        pl.kernel,
        out_type=jax.ShapeDtypeStruct((_P, _CT, _NTT, 8, _TI), jnp.float32),
        mesh=mesh,
        scratch_types=[
            pltpu.VMEM((_P, _TPW * _TI), jnp.int32),
            pltpu.VMEM((2, _TI, _D), jnp.float32),
            pltpu.VMEM((2, _D, _TI), jnp.float32),
            pltpu.SemaphoreType.DMA,
            pltpu.SemaphoreType.DMA,
            pltpu.SemaphoreType.DMA,
            pltpu.SemaphoreType.DMA,
        ],
        compiler_params=pltpu.CompilerParams(use_tc_tiling_on_sc=False,
                                             needs_layout_passes=False),
    )
    def gather_kernel(tid_hbm, table_hbm, out_hbm, idx_v, rows_v, obuf_v,
                      sg0, sg1, sw0, sw1):
        semg = (sg0, sg1)
        semw = (sw0, sw1)
        wid = lax.axis_index("s") * _NC + lax.axis_index("c")
        tt0 = wid * _TPW

        pltpu.sync_copy(
            tid_hbm.at[:, pl.ds(pl.multiple_of(tt0 * _TI, _TPW * _TI),
                                _TPW * _TI)],
            idx_v,
        )

        lanes = lax.iota(jnp.int32, 16)
        row_ids = [lanes + (16 * tg) for tg in range(_TI // 16)]

        def unit_parts(u):
            tti = u // _P
            return tti, u - tti * _P

        def fire_gather(u, b):
            tti, p = unit_parts(u)
            pltpu.async_copy(
                table_hbm.at[idx_v.at[p, pl.ds(pl.multiple_of(tti * _TI, _TI),
                                               _TI)]],
                rows_v.at[b],
                semg[b],
            )

        def drain_gather(b):
            pltpu.make_async_copy(
                table_hbm.at[pl.ds(0, _TI)], rows_v.at[b], semg[b]
            ).wait()

        def transpose(b):
            def col_body(c, carry):
                colv = jnp.full((16,), 0, jnp.int32) + c
                for tg in range(_TI // 16):
                    v = plsc.load_gather(rows_v.at[b], [row_ids[tg], colv])
                    obuf_v[b, c, pl.ds(tg * 16, 16)] = v
                return carry
            lax.fori_loop(0, _D, col_body, 0)

        def fire_writes(u, b):
            tti, p = unit_parts(u)
            tt = tt0 + tti
            for ct in range(_CT):
                pltpu.async_copy(
                    obuf_v.at[b].at[pl.ds(ct * 8, 8)],
                    out_hbm.at[p, ct, tt],
                    semw[b],
                )

        def wait_writes(b):
            for ct in range(_CT):
                pltpu.make_async_copy(
                    obuf_v.at[b].at[pl.ds(ct * 8, 8)],
                    out_hbm.at[0, ct, 0],
                    semw[b],
                ).wait()

        fire_gather(0, 0)

        @pl.loop(0, _UNITS, step=2)
        def _(u0):
            for bb in range(2):
                u = u0 + bb
                nb = 1 - bb

                @pl.when(u + 1 < _UNITS)
                def _():
                    fire_gather(u + 1, nb)

                drain_gather(bb)

                @pl.when(u0 >= 2 - bb)
                def _():
                    wait_writes(bb)

                transpose(bb)
                fire_writes(u, bb)

        wait_writes(0)
        wait_writes(1)

    return gather_kernel


def kernel(token_ids, W):
    tid = token_ids.T.astype(jnp.int32)
    ot = _make_gather()(tid, W)
    return ot.transpose(2, 4, 0, 1, 3).reshape(16384, _P, _D)


# speedup vs baseline: 1.7989x; 1.7989x over previous
import functools

import jax
import jax.numpy as jnp
from jax import lax
from jax.experimental import pallas as pl
from jax.experimental.pallas import tpu as pltpu
from jax.experimental.pallas import tpu_sc as plsc

_NC = 2
_NS = 16
_NW = _NC * _NS

_P = 50
_TI = 128
_NTT = 128
_D = 64
_CT = _D // 8
_TPW = _NTT // _NW
_UNITS = _TPW * _P


def _make_gather():
    mesh = plsc.VectorSubcoreMesh(core_axis_name="c", subcore_axis_name="s")

    @functools.partial(
        pl.kernel,
        out_type=jax.ShapeDtypeStruct((_P, _CT, _NTT, 8, _TI), jnp.float32),
        mesh=mesh,
        scratch_types=[
            pltpu.VMEM((_P, _TPW * _TI), jnp.int32),
            pltpu.VMEM((2, _TI, _D), jnp.float32),
            pltpu.VMEM((2, _D, _TI + 1), jnp.float32),
            pltpu.SemaphoreType.DMA,
            pltpu.SemaphoreType.DMA,
            pltpu.SemaphoreType.DMA,
            pltpu.SemaphoreType.DMA,
        ],
        compiler_params=pltpu.CompilerParams(use_tc_tiling_on_sc=False,
                                             needs_layout_passes=False),
    )
    def gather_kernel(tid_hbm, table_hbm, out_hbm, idx_v, rows_v, obuf_v,
                      sg0, sg1, sw0, sw1):
        semg = (sg0, sg1)
        semw = (sw0, sw1)
        wid = lax.axis_index("s") * _NC + lax.axis_index("c")
        tt0 = wid * _TPW

        pltpu.sync_copy(
            tid_hbm.at[:, pl.ds(pl.multiple_of(tt0 * _TI, _TPW * _TI),
                                _TPW * _TI)],
            idx_v,
        )

        lanes = lax.iota(jnp.int32, 16)
        c_ids = [lanes + (16 * cg) for cg in range(_D // 16)]

        def unit_parts(u):
            tti = u // _P
            return tti, u - tti * _P

        def fire_gather(u, b):
            tti, p = unit_parts(u)
            pltpu.async_copy(
                table_hbm.at[idx_v.at[p, pl.ds(pl.multiple_of(tti * _TI, _TI),
                                               _TI)]],
                rows_v.at[b],
                semg[b],
            )

        def drain_gather(b):
            pltpu.make_async_copy(
                table_hbm.at[pl.ds(0, _TI)], rows_v.at[b], semg[b]
            ).wait()

        def transpose(b):
            bsplat = jnp.full((16,), b, jnp.int32)

            def tok_body(t, carry):
                tsplat = jnp.full((16,), 0, jnp.int32) + t
                for cg in range(_D // 16):
                    v = rows_v[b, t, pl.ds(cg * 16, 16)]
                    plsc.store_scatter(obuf_v, [bsplat, c_ids[cg], tsplat], v)
                return carry
            lax.fori_loop(0, _TI, tok_body, 0)

        def fire_writes(u, b):
            tti, p = unit_parts(u)
            tt = tt0 + tti
            for ct in range(_CT):
                pltpu.async_copy(
                    obuf_v.at[b].at[pl.ds(ct * 8, 8), pl.ds(0, _TI)],
                    out_hbm.at[p, ct, tt],
                    semw[b],
                )

        def wait_writes(b):
            for ct in range(_CT):
                pltpu.make_async_copy(
                    obuf_v.at[b].at[pl.ds(ct * 8, 8), pl.ds(0, _TI)],
                    out_hbm.at[0, ct, 0],
                    semw[b],
                ).wait()

        fire_gather(0, 0)

        @pl.loop(0, _UNITS, step=2)
        def _(u0):
            for bb in range(2):
                u = u0 + bb
                nb = 1 - bb

                @pl.when(u + 1 < _UNITS)
                def _():
                    fire_gather(u + 1, nb)

                drain_gather(bb)

                @pl.when(u0 >= 2 - bb)
                def _():
                    wait_writes(bb)

                transpose(bb)
                fire_writes(u, bb)

        wait_writes(0)
        wait_writes(1)

    return gather_kernel


def kernel(token_ids, W):
    tid = token_ids.T.astype(jnp.int32)
    ot = _make_gather()(tid, W)
    return ot.transpose(2, 4, 0, 1, 3).reshape(16384, _P, _D)
